# baseline (speedup 1.0000x reference)
import functools

import jax
import jax.numpy as jnp
from jax import lax
from jax.experimental import pallas as pl
from jax.experimental.pallas import tpu as pltpu
from jax.experimental.pallas import tpu_sc as plsc

B, S, H, G = 16, 4096, 256, 512
NC, NS, L = 2, 16, 16
NW = NC * NS
SPT = NW // B
GT = G // SPT
K = 48
HV = H // L


def _at(ref, r):
    return ref[pl.ds(r, L)][0]


def _lower_bound(ids_v, target):
    def step(_, lohi):
        lo, hi = lohi
        mid = (lo + hi) // 2
        v = _at(ids_v, jnp.minimum(mid, S - 1))
        go = lo < hi
        less = v < target
        lo = jnp.where(go & less, mid + 1, lo)
        hi = jnp.where(go & (~less), mid, hi)
        return lo, hi
    lo, _ = lax.fori_loop(0, 13, step, (jnp.int32(0), jnp.int32(S)))
    return lo


def _seg_mean_body(feats_hbm, seg_hbm, out_hbm,
                   ids_v, stage, outbuf, cnt_v, sem0, sem1):
    c = lax.axis_index("c")
    s = lax.axis_index("s")
    w = s * NC + c
    b = w // SPT
    g0 = (w % SPT) * GT

    zero16 = jnp.zeros((L,), jnp.float32)

    def zrow(r, carry):
        for k in range(HV):
            outbuf[r, pl.ds(k * L, L)] = zero16
        cnt_v[r] = zero16
        return carry
    lax.fori_loop(0, GT, zrow, 0)

    pltpu.sync_copy(seg_hbm.at[b], ids_v.at[pl.ds(0, S)])
    ids_v[pl.ds(S, L)] = jnp.full((L,), G, jnp.int32)
    start = _lower_bound(ids_v, g0)
    end = _lower_bound(ids_v, g0 + GT)
    start_al = (start // 8) * 8
    nchunk = (end - start_al + K - 1) // K

    def fetch(off, buf):
        return pltpu.make_async_copy(
            feats_hbm.at[b, pl.ds(off, K)],
            stage.at[buf],
            sem0 if buf == 0 else sem1)

    def chunk_off(ci):
        return pl.multiple_of(jnp.minimum(start_al + ci * K, S - K), 8)

    @pl.when(nchunk > 0)
    def _():
        fetch(chunk_off(0), 0).start()

    def chunk_body(ci, carry):
        off = chunk_off(ci)
        buf = lax.rem(ci, 2)
        lo = jnp.maximum(start, start_al + ci * K)
        hi = jnp.minimum(end, start_al + ci * K + K)

        @pl.when(buf == 0)
        def _():
            fetch(off, 0).wait()

            @pl.when(ci + 1 < nchunk)
            def _():
                fetch(chunk_off(ci + 1), 1).start()

        @pl.when(buf == 1)
        def _():
            fetch(off, 1).wait()

            @pl.when(ci + 1 < nchunk)
            def _():
                fetch(chunk_off(ci + 1), 0).start()

        gs = _at(ids_v, lo) - g0
        ge = _at(ids_v, hi - 1) - g0

        def seg_body(g, rlo):
            lo_s = rlo
            hi_s = hi
            for _ in range(6):
                mid = (lo_s + hi_s) // 2
                v = _at(ids_v, jnp.minimum(mid, S - 1))
                go = lo_s < hi_s
                le = v <= g + g0
                lo_s = jnp.where(go & le, mid + 1, lo_s)
                hi_s = jnp.where(go & (~le), mid, hi_s)
            rhi = lo_s

            nrow = rhi - rlo
            base = rlo - off
            n4 = nrow - lax.rem(nrow, 4)

            def row4_body(q, acc):
                r = base + 4 * q
                for d in range(4):
                    acc = tuple(
                        acc[k] + stage[buf, r + d, pl.ds(k * L, L)]
                        for k in range(HV))
                return acc

            def row_body(r, acc):
                return tuple(
                    acc[k] + stage[buf, r, pl.ds(k * L, L)]
                    for k in range(HV))
            acc = lax.fori_loop(0, n4 // 4, row4_body,
                                tuple(zero16 for _ in range(HV)))
            acc = lax.fori_loop(base + n4, base + nrow, row_body, acc)
            for k in range(HV):
                plsc.addupdate(outbuf.at[g, pl.ds(k * L, L)], acc[k])
            cnt = (rhi - rlo).astype(jnp.float32)
            plsc.addupdate(cnt_v.at[g], jnp.full((L,), cnt, jnp.float32))
            return rhi
        lax.fori_loop(gs, ge + 1, seg_body, lo)
        return carry
    lax.fori_loop(0, nchunk, chunk_body, 0)

    def div_body(r, carry):
        wv = jnp.float32(1.0) / jnp.maximum(cnt_v[r], jnp.float32(1.0))
        for k in range(HV):
            outbuf[r, pl.ds(k * L, L)] = outbuf[r, pl.ds(k * L, L)] * wv
        return carry
    lax.fori_loop(0, GT, div_body, 0)
    pltpu.sync_copy(outbuf, out_hbm.at[pl.ds(b * G + g0, GT)])


_seg_mean = functools.partial(
    pl.kernel,
    mesh=plsc.VectorSubcoreMesh(core_axis_name="c", subcore_axis_name="s"),
    out_type=jax.ShapeDtypeStruct((B * G, H), jnp.float32),
    scratch_types=[
        pltpu.VMEM((S + L,), jnp.int32),
        pltpu.VMEM((2, K, H), jnp.float32),
        pltpu.VMEM((GT, H), jnp.float32),
        pltpu.VMEM((GT, L), jnp.float32),
        pltpu.SemaphoreType.DMA,
        pltpu.SemaphoreType.DMA,
    ],
)(_seg_mean_body)


def kernel(feats, segment_ids):
    seg = segment_ids.astype(jnp.int32)
    out = _seg_mean(feats, seg)
    return out.reshape(B, G, H)


# speedup vs baseline: 1.0106x; 1.0106x over previous
import functools

import jax
import jax.numpy as jnp
from jax import lax
from jax.experimental import pallas as pl
from jax.experimental.pallas import tpu as pltpu
from jax.experimental.pallas import tpu_sc as plsc

B, S, H, G = 16, 4096, 256, 512
NC, NS, L = 2, 16, 16
NW = NC * NS
SPT = NW // B
GT = G // SPT
K = 48
HV = H // L
SC = 512
LB0 = L


def _bounds_body(seg_ref, lbx_ref):
    ones = jnp.ones((1, SC), jnp.float32)
    giota = lax.broadcasted_iota(jnp.int32, (1, G), 1)
    acc = jnp.zeros((1, G), jnp.float32)
    for c in range(S // SC):
        ids = seg_ref[0, 0, pl.ds(c * SC, SC)].reshape(SC, 1)
        le = (ids <= giota).astype(jnp.float32)
        acc = acc + jnp.dot(ones, le,
                            preferred_element_type=jnp.float32)
    lbx_ref[0, 0, :] = acc[0, :].astype(jnp.int32)


_bounds = pl.pallas_call(
    _bounds_body,
    grid=(B,),
    in_specs=[pl.BlockSpec((1, 1, S), lambda b: (b, 0, 0))],
    out_specs=pl.BlockSpec((1, 1, G), lambda b: (b, 0, 0)),
    out_shape=jax.ShapeDtypeStruct((B, 1, G), jnp.int32),
)


def _at(ref, r):
    return ref[pl.ds(r, L)][0]


def _seg_mean_body(feats_hbm, seg_hbm, lbx_hbm, out_hbm,
                   ids_v, lb_v, stage, outbuf, cnt_v, sem0, sem1):
    c = lax.axis_index("c")
    s = lax.axis_index("s")
    w = s * NC + c
    b = w // SPT
    half = w % SPT
    g0 = half * GT

    zero16 = jnp.zeros((L,), jnp.float32)

    def zrow(r, carry):
        for k in range(HV):
            outbuf[r, pl.ds(k * L, L)] = zero16
        cnt_v[r] = zero16
        return carry
    lax.fori_loop(0, GT, zrow, 0)

    lb_v[pl.ds(0, L)] = jnp.zeros((L,), jnp.int32)
    pltpu.sync_copy(lbx_hbm.at[b, 0], lb_v.at[pl.ds(LB0, G)])

    pltpu.sync_copy(seg_hbm.at[b], ids_v.at[pl.ds(0, S)])
    ids_v[pl.ds(S, L)] = jnp.full((L,), G, jnp.int32)

    start = _at(lb_v, LB0 + g0 - 1)
    end = _at(lb_v, LB0 + g0 + GT - 1)
    start_al = (start // 8) * 8
    nchunk = (end - start_al + K - 1) // K

    def fetch(off, buf):
        return pltpu.make_async_copy(
            feats_hbm.at[b, pl.ds(off, K)],
            stage.at[buf],
            sem0 if buf == 0 else sem1)

    def chunk_off(ci):
        return pl.multiple_of(jnp.minimum(start_al + ci * K, S - K), 8)

    @pl.when(nchunk > 0)
    def _():
        fetch(chunk_off(0), 0).start()

    def chunk_body(ci, carry):
        off = chunk_off(ci)
        buf = lax.rem(ci, 2)
        lo = jnp.maximum(start, start_al + ci * K)
        hi = jnp.minimum(end, start_al + ci * K + K)

        @pl.when(buf == 0)
        def _():
            fetch(off, 0).wait()

            @pl.when(ci + 1 < nchunk)
            def _():
                fetch(chunk_off(ci + 1), 1).start()

        @pl.when(buf == 1)
        def _():
            fetch(off, 1).wait()

            @pl.when(ci + 1 < nchunk)
            def _():
                fetch(chunk_off(ci + 1), 0).start()

        gs = _at(ids_v, lo) - g0
        ge = _at(ids_v, hi - 1) - g0

        def seg_body(g, rlo):
            rhi = jnp.minimum(_at(lb_v, LB0 + g0 + g), hi)
            nrow = rhi - rlo
            base = rlo - off
            n4 = nrow - lax.rem(nrow, 4)

            def row4_body(q, acc):
                r = base + 4 * q
                for d in range(4):
                    acc = tuple(
                        acc[k] + stage[buf, r + d, pl.ds(k * L, L)]
                        for k in range(HV))
                return acc

            def row_body(r, acc):
                return tuple(
                    acc[k] + stage[buf, r, pl.ds(k * L, L)]
                    for k in range(HV))
            acc = lax.fori_loop(0, n4 // 4, row4_body,
                                tuple(zero16 for _ in range(HV)))
            acc = lax.fori_loop(base + n4, base + nrow, row_body, acc)
            for k in range(HV):
                plsc.addupdate(outbuf.at[g, pl.ds(k * L, L)], acc[k])
            cnt = nrow.astype(jnp.float32)
            plsc.addupdate(cnt_v.at[g], jnp.full((L,), cnt, jnp.float32))
            return rhi
        lax.fori_loop(gs, ge + 1, seg_body, lo)
        return carry
    lax.fori_loop(0, nchunk, chunk_body, 0)

    def div_body(r, carry):
        wv = jnp.float32(1.0) / jnp.maximum(cnt_v[r], jnp.float32(1.0))
        for k in range(HV):
            outbuf[r, pl.ds(k * L, L)] = outbuf[r, pl.ds(k * L, L)] * wv
        return carry
    lax.fori_loop(0, GT, div_body, 0)
    pltpu.sync_copy(outbuf, out_hbm.at[pl.ds(b * G + g0, GT)])


_seg_mean = functools.partial(
    pl.kernel,
    mesh=plsc.VectorSubcoreMesh(core_axis_name="c", subcore_axis_name="s"),
    out_type=jax.ShapeDtypeStruct((B * G, H), jnp.float32),
    scratch_types=[
        pltpu.VMEM((S + L,), jnp.int32),
        pltpu.VMEM((G + 2 * L,), jnp.int32),
        pltpu.VMEM((2, K, H), jnp.float32),
        pltpu.VMEM((GT, H), jnp.float32),
        pltpu.VMEM((GT, L), jnp.float32),
        pltpu.SemaphoreType.DMA,
        pltpu.SemaphoreType.DMA,
    ],
)(_seg_mean_body)


def kernel(feats, segment_ids):
    seg = segment_ids.astype(jnp.int32)
    lbx = _bounds(seg.reshape(B, 1, S))
    out = _seg_mean(feats, seg, lbx)
    return out.reshape(B, G, H)


# speedup vs baseline: 1.0272x; 1.0164x over previous
import functools

import jax
import jax.numpy as jnp
from jax import lax
from jax.experimental import pallas as pl
from jax.experimental.pallas import tpu as pltpu
from jax.experimental.pallas import tpu_sc as plsc

B, S, H, G = 16, 4096, 256, 512
NC, NS, L = 2, 16, 16
NW = NC * NS
SPT = NW // B
GT = G // SPT
K = 48
HV = H // L


def _at(ref, r):
    return ref[pl.ds(r, L)][0]


def _lower_bound(ids_v, target):
    def step(_, lohi):
        lo, hi = lohi
        mid = (lo + hi) // 2
        v = _at(ids_v, jnp.minimum(mid, S - 1))
        go = lo < hi
        less = v < target
        lo = jnp.where(go & less, mid + 1, lo)
        hi = jnp.where(go & (~less), mid, hi)
        return lo, hi
    lo, _ = lax.fori_loop(0, 13, step, (jnp.int32(0), jnp.int32(S)))
    return lo


def _seg_mean_body(feats_hbm, seg_hbm, out_hbm,
                   ids_v, stage, outbuf, cnt_v, sem0, sem1):
    c = lax.axis_index("c")
    s = lax.axis_index("s")
    w = s * NC + c
    b = w // SPT
    g0 = (w % SPT) * GT

    zero16 = jnp.zeros((L,), jnp.float32)

    def zrow(r, carry):
        for k in range(HV):
            outbuf[r, pl.ds(k * L, L)] = zero16
        cnt_v[r] = zero16
        return carry
    lax.fori_loop(0, GT, zrow, 0)

    pltpu.sync_copy(seg_hbm.at[b], ids_v.at[pl.ds(0, S)])
    ids_v[pl.ds(S, L)] = jnp.full((L,), G, jnp.int32)
    start = _lower_bound(ids_v, g0)
    end = _lower_bound(ids_v, g0 + GT)
    start_al = (start // 8) * 8
    nchunk = (end - start_al + K - 1) // K

    def fetch(off, buf):
        return pltpu.make_async_copy(
            feats_hbm.at[b, pl.ds(off, K)],
            stage.at[buf],
            sem0 if buf == 0 else sem1)

    def chunk_off(ci):
        return pl.multiple_of(jnp.minimum(start_al + ci * K, S - K), 8)

    @pl.when(nchunk > 0)
    def _():
        fetch(chunk_off(0), 0).start()

    def chunk_body(ci, carry):
        off = chunk_off(ci)
        buf = lax.rem(ci, 2)
        lo = jnp.maximum(start, start_al + ci * K)
        hi = jnp.minimum(end, start_al + ci * K + K)

        @pl.when(buf == 0)
        def _():
            fetch(off, 0).wait()

            @pl.when(ci + 1 < nchunk)
            def _():
                fetch(chunk_off(ci + 1), 1).start()

        @pl.when(buf == 1)
        def _():
            fetch(off, 1).wait()

            @pl.when(ci + 1 < nchunk)
            def _():
                fetch(chunk_off(ci + 1), 0).start()

        gs = _at(ids_v, lo) - g0
        ge = _at(ids_v, hi - 1) - g0

        def seg_body(g, rlo):
            lo_s = rlo
            hi_s = hi
            for _ in range(6):
                mid = (lo_s + hi_s) // 2
                v = _at(ids_v, jnp.minimum(mid, S - 1))
                go = lo_s < hi_s
                le = v <= g + g0
                lo_s = jnp.where(go & le, mid + 1, lo_s)
                hi_s = jnp.where(go & (~le), mid, hi_s)
            rhi = lo_s

            def row_body(r, acc):
                return tuple(
                    acc[k] + stage[buf, r, pl.ds(k * L, L)]
                    for k in range(HV))
            acc = lax.fori_loop(rlo - off, rhi - off, row_body,
                                tuple(zero16 for _ in range(HV)))
            for k in range(HV):
                plsc.addupdate(outbuf.at[g, pl.ds(k * L, L)], acc[k])
            cnt = (rhi - rlo).astype(jnp.float32)
            plsc.addupdate(cnt_v.at[g], jnp.full((L,), cnt, jnp.float32))
            return rhi
        lax.fori_loop(gs, ge + 1, seg_body, lo)
        return carry
    lax.fori_loop(0, nchunk, chunk_body, 0)

    def div_body(r, carry):
        wv = jnp.float32(1.0) / jnp.maximum(cnt_v[r], jnp.float32(1.0))
        for k in range(HV):
            outbuf[r, pl.ds(k * L, L)] = outbuf[r, pl.ds(k * L, L)] * wv
        return carry
    lax.fori_loop(0, GT, div_body, 0)
    pltpu.sync_copy(outbuf, out_hbm.at[pl.ds(b * G + g0, GT)])


_seg_mean = functools.partial(
    pl.kernel,
    mesh=plsc.VectorSubcoreMesh(core_axis_name="c", subcore_axis_name="s"),
    out_type=jax.ShapeDtypeStruct((B * G, H), jnp.float32),
    scratch_types=[
        pltpu.VMEM((S + L,), jnp.int32),
        pltpu.VMEM((2, K, H), jnp.float32),
        pltpu.VMEM((GT, H), jnp.float32),
        pltpu.VMEM((GT, L), jnp.float32),
        pltpu.SemaphoreType.DMA,
        pltpu.SemaphoreType.DMA,
    ],
)(_seg_mean_body)


def kernel(feats, segment_ids):
    seg = segment_ids.astype(jnp.int32)
    out = _seg_mean(feats, seg)
    return out.reshape(B, G, H)


# speedup vs baseline: 1.0365x; 1.0090x over previous
import functools

import jax
import jax.numpy as jnp
from jax import lax
from jax.experimental import pallas as pl
from jax.experimental.pallas import tpu as pltpu
from jax.experimental.pallas import tpu_sc as plsc

B, S, H, G = 16, 4096, 256, 512
NC, NS, L = 2, 16, 16
NW = NC * NS
SPT = NW // B
GT = G // SPT
K = 48
HV = H // L


def _at(ref, r):
    return ref[pl.ds(r, L)][0]


def _lower_bound(ids_v, target):
    def step(_, lohi):
        lo, hi = lohi
        mid = (lo + hi) // 2
        v = _at(ids_v, jnp.minimum(mid, S - 1))
        go = lo < hi
        less = v < target
        lo = jnp.where(go & less, mid + 1, lo)
        hi = jnp.where(go & (~less), mid, hi)
        return lo, hi
    lo, _ = lax.fori_loop(0, 13, step, (jnp.int32(0), jnp.int32(S)))
    return lo


def _seg_mean_body(feats_hbm, seg_hbm, out_hbm,
                   ids_v, stage, outbuf, cnt_v, sem0, sem1):
    c = lax.axis_index("c")
    s = lax.axis_index("s")
    w = s * NC + c
    b = w // SPT
    g0 = (w % SPT) * GT

    zero16 = jnp.zeros((L,), jnp.float32)

    ids_v[pl.ds(S, L)] = jnp.full((L,), G, jnp.int32)
    ids_copy = pltpu.make_async_copy(
        seg_hbm.at[b], ids_v.at[pl.ds(0, S)], sem0)
    ids_copy.start()

    def zrow(r, carry):
        for k in range(HV):
            outbuf[r, pl.ds(k * L, L)] = zero16
        cnt_v[r] = zero16
        return carry
    lax.fori_loop(0, GT, zrow, 0)

    ids_copy.wait()
    start = _lower_bound(ids_v, g0)
    end = _lower_bound(ids_v, g0 + GT)
    start_al = (start // 8) * 8
    nchunk = (end - start_al + K - 1) // K

    def fetch(off, buf):
        return pltpu.make_async_copy(
            feats_hbm.at[b, pl.ds(off, K)],
            stage.at[buf],
            sem0 if buf == 0 else sem1)

    def chunk_off(ci):
        return pl.multiple_of(jnp.minimum(start_al + ci * K, S - K), 8)

    @pl.when(nchunk > 0)
    def _():
        fetch(chunk_off(0), 0).start()

    def chunk_body(ci, carry):
        off = chunk_off(ci)
        buf = lax.rem(ci, 2)
        lo = jnp.maximum(start, start_al + ci * K)
        hi = jnp.minimum(end, start_al + ci * K + K)

        @pl.when(buf == 0)
        def _():
            fetch(off, 0).wait()

            @pl.when(ci + 1 < nchunk)
            def _():
                fetch(chunk_off(ci + 1), 1).start()

        @pl.when(buf == 1)
        def _():
            fetch(off, 1).wait()

            @pl.when(ci + 1 < nchunk)
            def _():
                fetch(chunk_off(ci + 1), 0).start()

        gs = _at(ids_v, lo) - g0
        ge = _at(ids_v, hi - 1) - g0

        def seg_body(g, rlo):
            lo_s = rlo
            hi_s = hi
            for _ in range(6):
                mid = (lo_s + hi_s) // 2
                v = _at(ids_v, jnp.minimum(mid, S - 1))
                go = lo_s < hi_s
                le = v <= g + g0
                lo_s = jnp.where(go & le, mid + 1, lo_s)
                hi_s = jnp.where(go & (~le), mid, hi_s)
            rhi = lo_s

            def row_body(r, acc):
                return tuple(
                    acc[k] + stage[buf, r, pl.ds(k * L, L)]
                    for k in range(HV))
            acc = lax.fori_loop(rlo - off, rhi - off, row_body,
                                tuple(zero16 for _ in range(HV)))
            for k in range(HV):
                plsc.addupdate(outbuf.at[g, pl.ds(k * L, L)], acc[k])
            cnt = (rhi - rlo).astype(jnp.float32)
            plsc.addupdate(cnt_v.at[g], jnp.full((L,), cnt, jnp.float32))
            return rhi
        lax.fori_loop(gs, ge + 1, seg_body, lo)
        return carry
    lax.fori_loop(0, nchunk, chunk_body, 0)

    def div_body(r, carry):
        wv = jnp.float32(1.0) / jnp.maximum(cnt_v[r], jnp.float32(1.0))
        for k in range(HV):
            outbuf[r, pl.ds(k * L, L)] = outbuf[r, pl.ds(k * L, L)] * wv
        return carry
    lax.fori_loop(0, GT, div_body, 0)
    pltpu.sync_copy(outbuf, out_hbm.at[pl.ds(b * G + g0, GT)])


_seg_mean = functools.partial(
    pl.kernel,
    mesh=plsc.VectorSubcoreMesh(core_axis_name="c", subcore_axis_name="s"),
    out_type=jax.ShapeDtypeStruct((B * G, H), jnp.float32),
    scratch_types=[
        pltpu.VMEM((S + L,), jnp.int32),
        pltpu.VMEM((2, K, H), jnp.float32),
        pltpu.VMEM((GT, H), jnp.float32),
        pltpu.VMEM((GT, L), jnp.float32),
        pltpu.SemaphoreType.DMA,
        pltpu.SemaphoreType.DMA,
    ],
)(_seg_mean_body)


def kernel(feats, segment_ids):
    seg = segment_ids.astype(jnp.int32)
    out = _seg_mean(feats, seg)
    return out.reshape(B, G, H)
